# baseline (speedup 1.0000x reference)
import functools

import jax
import jax.numpy as jnp
from jax import lax
from jax.experimental import pallas as pl
from jax.experimental.pallas import tpu as pltpu
from jax.experimental.pallas import tpu_sc as plsc

D_MODEL = 512
FONT_X = 128
FONT_Y = 128
VOCAB = 512
BATCH = 32
SEQ = 2048

NUM_CORES = 2
NUM_SUBCORES = 16
NUM_WORKERS = NUM_CORES * NUM_SUBCORES
LANES = 16

B_SC = 16
B_TC = BATCH - B_SC
S_OWN = SEQ // NUM_WORKERS
ROWS = 32
NBUF = 4
STEPS = B_SC * S_OWN // ROWS

T_CHUNK = 512


def _fuse_body(tok_ref, cxm_ref, cym_ref, cx1_ref, cy1_ref, out_ref, obf_ref):
    out_ref[0:FONT_X, :] = tok_ref[0:FONT_X, :] + cxm_ref[:, :] + cy1_ref[:, :]
    out_ref[FONT_X:FONT_X + FONT_Y, :] = (
        tok_ref[FONT_X:FONT_X + FONT_Y, :] + cx1_ref[:, :] + cym_ref[:, :])
    out_ref[FONT_X + FONT_Y:VOCAB, :] = (
        tok_ref[FONT_X + FONT_Y:VOCAB, :] + cx1_ref[:, :] + cy1_ref[:, :])
    obf_ref[:, :] = out_ref[:, :].astype(jnp.bfloat16)


def _build_fused(token_table, coord_x_table, coord_y_table):
    cxm = coord_x_table[1:FONT_X + 1]
    cym = coord_y_table[1:FONT_Y + 1]
    cx1 = coord_x_table[1:2]
    cy1 = coord_y_table[1:2]
    return pl.pallas_call(
        _fuse_body,
        out_shape=(jax.ShapeDtypeStruct((VOCAB, D_MODEL), jnp.float32),
                   jax.ShapeDtypeStruct((VOCAB, D_MODEL), jnp.bfloat16)),
    )(token_table, cxm, cym, cx1, cy1)


def _onehot_body(tok_ref, fused_ref, pos_ref, out_ref):
    tok = tok_ref[0, 0, :]
    onehot = (tok[:, None] ==
              lax.broadcasted_iota(jnp.int32, (T_CHUNK, VOCAB), 1))
    acc = jnp.dot(onehot.astype(jnp.bfloat16), fused_ref[:, :],
                  preferred_element_type=jnp.float32)
    out_ref[:, :] = acc + pos_ref[:, :]


def _tc_lookup(tok_tc, fused_bf16, pos_table):
    n_chunks = B_TC * SEQ // T_CHUNK
    per_seq = SEQ // T_CHUNK
    return pl.pallas_call(
        _onehot_body,
        grid=(n_chunks,),
        in_specs=[
            pl.BlockSpec((1, 1, T_CHUNK), lambda c: (c, 0, 0)),
            pl.BlockSpec((VOCAB, D_MODEL), lambda c: (0, 0)),
            pl.BlockSpec((T_CHUNK, D_MODEL), lambda c: (c % per_seq, 0)),
        ],
        out_specs=pl.BlockSpec((T_CHUNK, D_MODEL), lambda c: (c, 0)),
        out_shape=jax.ShapeDtypeStruct((B_TC * SEQ, D_MODEL), jnp.float32),
    )(tok_tc.reshape(n_chunks, 1, T_CHUNK), fused_bf16, pos_table)


def _lookup_body(tok_hbm, fused_hbm, pos_hbm, out_hbm,
                 idx_all, pos_v, rbufs, gsems, wsems, isem):
    wid = lax.axis_index("s") * NUM_CORES + lax.axis_index("c")
    s0 = wid * S_OWN
    pltpu.sync_copy(pos_hbm.at[pl.ds(s0, S_OWN)], pos_v)
    icps = [pltpu.async_copy(tok_hbm.at[pl.ds(b * SEQ + s0, S_OWN)],
                             idx_all.at[b], isem)
            for b in range(B_SC)]
    for c in icps:
        c.wait()

    def gather_start(t, p):
        idx_ref = idx_all.at[t // 2, pl.ds((t % 2) * ROWS, ROWS)]
        pltpu.async_copy(fused_hbm.at[idx_ref], rbufs[p], gsems[p])

    def wait_gather(p):
        pltpu.make_async_copy(
            fused_hbm.at[idx_all.at[0, pl.ds(0, ROWS)]],
            rbufs[p], gsems[p]).wait()

    def write_start(t, p):
        off = (t // 2) * SEQ + s0 + (t % 2) * ROWS
        pltpu.async_copy(rbufs[p], out_hbm.at[pl.ds(off, ROWS)], wsems[p])

    def wait_write(p):
        pltpu.make_async_copy(
            rbufs[p], out_hbm.at[pl.ds(0, ROWS)], wsems[p]).wait()

    gather_start(0, 0)

    def outer_body(g, carry):
        for ph in range(NBUF):
            t = g * NBUF + ph
            p1 = (ph + 1) % NBUF
            if ph < NBUF - 1:
                @pl.when(g >= 1)
                def _():
                    wait_write(p1)
                gather_start(t + 1, p1)
            else:
                wait_write(p1)

                @pl.when(g < STEPS // NBUF - 1)
                def _():
                    gather_start(t + 1, p1)
            wait_gather(ph)
            half = ph % 2

            def row_body(j, c2, _ph=ph, _half=half):
                for k in range(D_MODEL // LANES):
                    sl = pl.ds(k * LANES, LANES)
                    plsc.addupdate(rbufs[_ph].at[j, sl],
                                   pos_v[_half * ROWS + j, sl])
                return c2

            lax.fori_loop(0, ROWS, row_body, 0)
            write_start(t, ph)
        return carry

    lax.fori_loop(0, STEPS // NBUF, outer_body, 0)
    for p in range(1, NBUF):
        wait_write(p)


_lookup = functools.partial(
    pl.kernel,
    out_type=jax.ShapeDtypeStruct((B_SC * SEQ, D_MODEL), jnp.float32),
    mesh=plsc.VectorSubcoreMesh(
        core_axis_name="c", subcore_axis_name="s",
        num_cores=NUM_CORES, num_subcores=NUM_SUBCORES),
    scratch_types=[
        pltpu.VMEM((B_SC, S_OWN), jnp.int32),
        pltpu.VMEM((S_OWN, D_MODEL), jnp.float32),
        [pltpu.VMEM((ROWS, D_MODEL), jnp.float32) for _ in range(NBUF)],
        [pltpu.SemaphoreType.DMA for _ in range(NBUF)],
        [pltpu.SemaphoreType.DMA for _ in range(NBUF)],
        pltpu.SemaphoreType.DMA,
    ],
)(_lookup_body)


def kernel(font_tokens, token_table, coord_x_table, coord_y_table, pos_table):
    fused, fused_bf16 = _build_fused(token_table, coord_x_table, coord_y_table)
    tokens = font_tokens.astype(jnp.int32).reshape(BATCH * SEQ)
    out_sc = _lookup(tokens[:B_SC * SEQ], fused, pos_table)
    out_tc = _tc_lookup(tokens[B_SC * SEQ:], fused_bf16, pos_table)
    out = jnp.concatenate([out_sc, out_tc], axis=0)
    return out.reshape(BATCH, SEQ, D_MODEL)
